# baseline (speedup 1.0000x reference)
import functools

import jax
import jax.numpy as jnp
from jax import lax
from jax.experimental import pallas as pl
from jax.experimental.pallas import tpu as pltpu
from jax.experimental.pallas import tpu_sc as plsc

_TN = 2048
_K = 4
_NBUF = 4


def _h_body(emb_ref, W1_ref, b1_ref, h_ref):
    h_ref[...] = jnp.tanh(
        jnp.dot(emb_ref[...], W1_ref[...],
                preferred_element_type=jnp.float32) + b1_ref[...])


def _chunk_body(h_ref, W2_ref, b2_ref, p_ref):
    p_ref[...] = jnp.dot(h_ref[...], W2_ref[...],
                         preferred_element_type=jnp.float32) + b2_ref[...]


def _project_h(emb, W1, b1):
    V, D = emb.shape
    H = W1.shape[1]
    return pl.pallas_call(
        _h_body,
        in_specs=[
            pl.BlockSpec((V, D), lambda: (0, 0)),
            pl.BlockSpec((D, H), lambda: (0, 0)),
            pl.BlockSpec((1, H), lambda: (0, 0)),
        ],
        out_specs=pl.BlockSpec((V, H), lambda: (0, 0)),
        out_shape=jax.ShapeDtypeStruct((V, H), jnp.float32),
    )(emb, W1, b1.reshape(1, H))


def _project_chunk(h, W2, b2r, c0, C):
    V, H = h.shape
    D = W2.shape[0]
    t0 = c0 // _TN
    return pl.pallas_call(
        _chunk_body,
        grid=(C // _TN,),
        in_specs=[
            pl.BlockSpec((V, H), lambda i: (0, 0)),
            pl.BlockSpec((D, _TN), lambda i: (0, t0 + i)),
            pl.BlockSpec((1, _TN), lambda i: (0, t0 + i)),
        ],
        out_specs=pl.BlockSpec((V, _TN), lambda i: (0, i)),
        out_shape=jax.ShapeDtypeStruct((V, C), jnp.float32),
    )(h, W2, b2r)


@functools.lru_cache(maxsize=None)
def _make_sc_expand(V, N, B, C, c0, alloc_out):
    info = plsc.get_sparse_core_info()
    NC = info.num_cores
    NW = NC * info.num_subcores
    TPW = B // NW
    mesh = plsc.VectorSubcoreMesh(core_axis_name="c", subcore_axis_name="s")

    def body(p_hbm, idx_hbm, out_hbm, idx_v,
             buf0, buf1, buf2, buf3,
             semg0, semg1, semg2, semg3,
             sems0, sems1, sems2, sems3):
        wid = lax.axis_index("s") * NC + lax.axis_index("c")
        base_t = wid * TPW
        bufs = (buf0, buf1, buf2, buf3)
        semg = (semg0, semg1, semg2, semg3)
        sems = (sems0, sems1, sems2, sems3)
        pltpu.sync_copy(idx_hbm.at[pl.ds(base_t, TPW)], idx_v)
        outv = out_hbm.at[:, pl.ds(c0, C)]

        def gather(j):
            pltpu.async_copy(p_hbm.at[idx_v.at[j]], bufs[j % _NBUF],
                             semg[j % _NBUF])

        def store_desc(j):
            return pltpu.make_async_copy(
                bufs[j % _NBUF], outv.at[pl.ds(base_t + j, 1)],
                sems[j % _NBUF])

        gather(0)
        gather(1)
        for j in range(TPW):
            b = j % _NBUF
            pltpu.make_async_copy(p_hbm.at[idx_v.at[j]], bufs[b],
                                  semg[b]).wait()
            store_desc(j).start()
            nj = j + 2
            if nj < TPW:
                if nj >= _NBUF:
                    store_desc(nj - _NBUF).wait()
                gather(nj)
        store_desc(TPW - 2).wait()
        store_desc(TPW - 1).wait()

    scratch = (
        [pltpu.VMEM((TPW, 1), jnp.int32)]
        + [pltpu.VMEM((1, C), jnp.float32)] * _NBUF
        + [pltpu.SemaphoreType.DMA] * (2 * _NBUF)
    )
    out_type = jax.ShapeDtypeStruct((B, N), jnp.float32) if alloc_out else ()
    return pl.kernel(body, out_type=out_type, mesh=mesh,
                     scratch_types=scratch)


def kernel(prefix, emb, W1, b1, W2, b2):
    B, L = prefix.shape
    T = B * L
    V, D = emb.shape
    N = W2.shape[1]
    C = N // _K
    b2r = b2.reshape(1, N)
    idx = prefix.reshape(T, 1).astype(jnp.int32)

    h = _project_h(emb, W1, b1)
    p_chunks = [_project_chunk(h, W2, b2r, c * C, C) for c in range(_K)]

    out0 = _make_sc_expand(V, N, T, C, 0, True)(p_chunks[0], idx)
    out_ref = jax.new_ref(out0)
    for c in range(1, _K):
        _make_sc_expand(V, N, T, C, c * C, False)(
            p_chunks[c], idx, out_ref)
    return out_ref[...].reshape(B, L, N)


# speedup vs baseline: 1.7264x; 1.7264x over previous
import jax
import jax.numpy as jnp
from jax.experimental import pallas as pl
from jax.experimental.pallas import tpu as pltpu

_TN = 2048


def _body(idx_ref, emb_ref, W1_ref, b1_ref, W2_ref, b2_ref, out_ref,
          h_ref, oh_ref):
    step = pl.program_id(0)

    @pl.when(step == 0)
    def _prologue():
        h_ref[...] = jnp.tanh(
            jnp.dot(emb_ref[...], W1_ref[...],
                    preferred_element_type=jnp.float32) + b1_ref[...])
        T, V = oh_ref.shape
        iota = jax.lax.broadcasted_iota(jnp.int32, (T, V), 1)
        oh_ref[...] = (idx_ref[...] == iota).astype(jnp.float32)

    p = jnp.dot(h_ref[...], W2_ref[...],
                preferred_element_type=jnp.float32) + b2_ref[...]
    out_ref[...] = jnp.dot(oh_ref[...], p,
                           preferred_element_type=jnp.float32)


def kernel(prefix, emb, W1, b1, W2, b2):
    B, L = prefix.shape
    T = B * L
    V, D = emb.shape
    H = W1.shape[1]
    N = W2.shape[1]
    idx = prefix.reshape(T, 1).astype(jnp.int32)
    b1r = b1.reshape(1, H)
    b2r = b2.reshape(1, N)
    grid = N // _TN

    out = pl.pallas_call(
        _body,
        grid=(grid,),
        in_specs=[
            pl.BlockSpec((T, 1), lambda i: (0, 0)),
            pl.BlockSpec((V, D), lambda i: (0, 0)),
            pl.BlockSpec((D, H), lambda i: (0, 0)),
            pl.BlockSpec((1, H), lambda i: (0, 0)),
            pl.BlockSpec((D, _TN), lambda i: (0, i)),
            pl.BlockSpec((1, _TN), lambda i: (0, i)),
        ],
        out_specs=pl.BlockSpec((T, _TN), lambda i: (0, i)),
        out_shape=jax.ShapeDtypeStruct((T, N), jnp.float32),
        scratch_shapes=[
            pltpu.VMEM((V, H), jnp.float32),
            pltpu.VMEM((T, V), jnp.float32),
        ],
    )(idx, emb, W1, b1r, W2, b2r)
    return out.reshape(B, L, N)
